# baseline (speedup 1.0000x reference)
import functools

import jax
import jax.numpy as jnp
from jax import lax
from jax.experimental import pallas as pl
from jax.experimental.pallas import tpu as pltpu
from jax.experimental.pallas import tpu_sc as plsc

_NBUF = 4
_W = 128


def kernel(x, table):
    B, S = x.shape
    V, D = table.shape
    offs = (0, S - _W)

    info = plsc.get_sparse_core_info()
    NC, NS = info.num_cores, info.num_subcores
    NW = NC * NS

    bat_per_w = B // NW
    steady = bat_per_w - _NBUF
    assert B % NW == 0 and _W <= S <= 2 * _W and (S - _W) % 8 == 0
    assert steady % _NBUF == 0 and steady >= 0

    xi = x.astype(jnp.int32)

    mesh = plsc.VectorSubcoreMesh(core_axis_name="c", subcore_axis_name="s")

    @functools.partial(
        pl.kernel,
        mesh=mesh,
        compiler_params=pltpu.CompilerParams(use_tc_tiling_on_sc=False),
        out_type=jax.ShapeDtypeStruct((B, S, D), jnp.float32),
        scratch_types=[
            pltpu.VMEM((bat_per_w, S), jnp.int32),
            pltpu.VMEM((_NBUF, S, D), jnp.float32),
            [pltpu.SemaphoreType.DMA] * _NBUF,
        ],
    )
    def emb(idx_hbm, table_hbm, out_hbm, idx_all, rows_v, gsems):
        wid = lax.axis_index("s") * NC + lax.axis_index("c")
        bat_base = wid * bat_per_w

        pltpu.sync_copy(idx_hbm.at[pl.ds(bat_base, bat_per_w)], idx_all)

        def fire(g, b):
            for off in offs:
                pltpu.async_copy(
                    table_hbm.at[idx_all.at[g, pl.ds(off, _W)]],
                    rows_v.at[b, pl.ds(off, _W)],
                    gsems[b],
                )

        def drain(g, b):
            for off in offs:
                pltpu.make_async_copy(
                    out_hbm.at[0, pl.ds(off, _W)],
                    rows_v.at[b, pl.ds(off, _W)],
                    gsems[b],
                ).wait()
            pltpu.sync_copy(rows_v.at[b], out_hbm.at[bat_base + g])

        for b in range(_NBUF):
            fire(b, b)

        def body(i, carry):
            gg = i * _NBUF
            for b in range(_NBUF):
                drain(gg + b, b)
                fire(gg + b + _NBUF, b)
            return carry

        lax.fori_loop(0, steady // _NBUF, body, 0)

        for b in range(_NBUF):
            drain(steady + b, b)

    return emb(xi, table)
